# baseline (speedup 1.0000x reference)
import functools

import jax
import jax.numpy as jnp
from jax import lax
from jax.experimental import pallas as pl
from jax.experimental.pallas import tpu as pltpu
from jax.experimental.pallas import tpu_sc as plsc

WORD_DIM = 128
POS_DIM = 16
OUT_DIM = WORD_DIM + 3 * POS_DIM
NUM_CORES = 2
NUM_SUBCORES = 16
NUM_WORKERS = NUM_CORES * NUM_SUBCORES
BB = 128
NBUF = 2

_DIMS = (WORD_DIM, POS_DIM, POS_DIM, POS_DIM)


def _build(B: int, L: int):
    n_bt = B // BB
    n_units = L * n_bt
    units_per_w = n_units // NUM_WORKERS
    DT = OUT_DIM // 8
    WT = WORD_DIM // 8
    mesh = plsc.VectorSubcoreMesh(
        core_axis_name="c", subcore_axis_name="s",
        num_cores=NUM_CORES, num_subcores=NUM_SUBCORES)

    BBP = BB + 8
    NI = 4
    NW = 4
    scratch = (
        [pltpu.VMEM((BB,), jnp.int32) for _ in range(4 * NI)]
        + [pltpu.VMEM((BB, WORD_DIM), jnp.float32) for _ in range(NW)]
        + [pltpu.VMEM((BB, POS_DIM), jnp.float32) for _ in range(3 * NBUF)]
        + [pltpu.VMEM((DT, 8, BBP), jnp.float32) for _ in range(NBUF)]
        + [pltpu.SemaphoreType.DMA for _ in range(NI + NW + 2 * NBUF)]
    )

    @functools.partial(
        pl.kernel,
        mesh=mesh,
        compiler_params=pltpu.CompilerParams(use_tc_tiling_on_sc=False,
                                             needs_layout_passes=False),
        out_type=jax.ShapeDtypeStruct((L, DT, n_bt, 8, BB), jnp.float32),
        scratch_types=scratch,
    )
    def fused_embed(word_h, p0_h, p1_h, p2_h, wt_h, t0_h, t1_h, t2_h,
                    out_h, *scr):
        idx_bufs = [scr[4 * q:4 * q + 4] for q in range(NI)]
        o = 4 * NI
        wbufs = scr[o:o + NW]
        o += NW
        pbufs = [scr[o + 3 * k:o + 3 * k + 3] for k in range(NBUF)]
        o += 3 * NBUF
        tbufs = scr[o:o + NBUF]
        o += NBUF
        sem_i = scr[o:o + NI]
        sem_w = scr[o + NI:o + NI + NW]
        sem_p = scr[o + NI + NW:o + NI + NW + NBUF]
        sem_o = scr[o + NI + NW + NBUF:o + NI + NW + 2 * NBUF]
        srcs = (word_h, p0_h, p1_h, p2_h)
        ptabs = (t0_h, t1_h, t2_h)

        wid = lax.axis_index("s") * NUM_CORES + lax.axis_index("c")
        u0 = wid * units_per_w
        iota = lax.iota(jnp.int32, 16)
        dvecs = [j * 16 + iota for j in range(OUT_DIM // 16)]
        i0s = [d // 8 for d in dvecs]
        i1s = [d % 8 for d in dvecs]

        def idx_issue(u, q):
            base = jnp.minimum(u, u0 + units_per_w - 1) * BB
            for s, d in zip(srcs, idx_bufs[q]):
                pltpu.async_copy(s.at[pl.ds(base, BB)], d, sem_i[q])

        def idx_wait(q):
            for s, d in zip(srcs, idx_bufs[q]):
                pltpu.make_async_copy(s.at[pl.ds(0, BB)], d, sem_i[q]).wait()

        def word_issue(kw, q):
            pltpu.async_copy(wt_h.at[idx_bufs[q][0]], wbufs[kw], sem_w[kw])

        def word_wait(kw, q):
            pltpu.make_async_copy(wt_h.at[idx_bufs[q][0]], wbufs[kw],
                                  sem_w[kw]).wait()

        def pos_issue(k, q):
            for tab, iv, b in zip(ptabs, idx_bufs[q][1:], pbufs[k]):
                pltpu.async_copy(tab.at[iv], b, sem_p[k])

        def pos_wait(k, q):
            for tab, iv, b in zip(ptabs, idx_bufs[q][1:], pbufs[k]):
                pltpu.make_async_copy(tab.at[iv], b, sem_p[k]).wait()

        HB = BB // 2

        def write_issue(u, k, half):
            l, bt = u // n_bt, u % n_bt
            pltpu.async_copy(tbufs[k].at[:, :, pl.ds(half * HB, HB)],
                             out_h.at[l, :, bt, :, pl.ds(half * HB, HB)],
                             sem_o[k])

        def write_wait(k):
            for half in range(2):
                pltpu.make_async_copy(tbufs[k].at[:, :, pl.ds(half * HB, HB)],
                                      out_h.at[0, :, 0, :, pl.ds(half * HB, HB)],
                                      sem_o[k]).wait()

        def transpose_half(kw, k, half):
            bw = wbufs[kw]
            b0, b1, b2 = pbufs[k]
            tb = tbufs[k]
            nw = WORD_DIM // 16

            def t_body(th, carry):
                for dt in range(2):
                    t = th * 2 + dt
                    tvec = jnp.full((16,), t, jnp.int32)
                    for j in range(nw):
                        vals = bw[t, pl.ds(j * 16, 16)]
                        plsc.store_scatter(tb, [i0s[j], i1s[j], tvec], vals)
                    for pi, bp in enumerate((b0, b1, b2)):
                        j = nw + pi
                        vals = bp[t, pl.ds(0, POS_DIM)]
                        plsc.store_scatter(tb, [i0s[j], i1s[j], tvec], vals)
                return carry

            lax.fori_loop(half * HB // 2, (half + 1) * HB // 2, t_body, 0)

        for q in range(NI):
            idx_issue(u0 + q, q)
        for u in range(NBUF):
            idx_wait(u)
            word_issue(u, u)
            pos_issue(u, u)

        def body(t, carry):
            u_base = u0 + t * NI
            for j in range(NI):
                u = u_base + j
                k = j % NBUF
                word_wait(j, j)
                pos_wait(k, j)
                idx_wait((j + NBUF) % NI)
                word_issue((j + NBUF) % NW, (j + NBUF) % NI)
                if j >= NBUF:
                    write_wait(k)
                else:
                    @pl.when(t > 0)
                    def _(k=k):
                        write_wait(k)
                transpose_half(j, k, 0)
                write_issue(u, k, 0)
                transpose_half(j, k, 1)
                idx_issue(u + NI, j)
                write_issue(u, k, 1)
                pos_issue(k, (j + NBUF) % NI)
            return carry

        lax.fori_loop(0, units_per_w // NI, body, 0)
        for q in range(NBUF):
            idx_wait((q + NBUF) % NI)
            word_wait(q, q)
            pos_wait(q, q)
            write_wait(q)

    return fused_embed


def kernel(word, pos0, pos1, pos2, word_table, pos0_table, pos1_table, pos2_table):
    B, L = word.shape
    fused = _build(B, L)
    X = fused(word.T.reshape(B * L), pos0.T.reshape(B * L),
              pos1.T.reshape(B * L), pos2.T.reshape(B * L),
              word_table, pos0_table, pos1_table, pos2_table)
    return X.transpose(2, 4, 0, 1, 3).reshape(B, L, OUT_DIM)


# speedup vs baseline: 1.0143x; 1.0143x over previous
import functools

import jax
import jax.numpy as jnp
from jax import lax
from jax.experimental import pallas as pl
from jax.experimental.pallas import tpu as pltpu
from jax.experimental.pallas import tpu_sc as plsc

WORD_DIM = 128
POS_DIM = 16
OUT_DIM = WORD_DIM + 3 * POS_DIM
NUM_CORES = 2
NUM_SUBCORES = 16
NUM_WORKERS = NUM_CORES * NUM_SUBCORES
BB = 128
NBUF = 2

_DIMS = (WORD_DIM, POS_DIM, POS_DIM, POS_DIM)


def _build(B: int, L: int):
    n_bt = B // BB
    n_units = L * n_bt
    units_per_w = n_units // NUM_WORKERS
    DT = OUT_DIM // 8
    WT = WORD_DIM // 8
    mesh = plsc.VectorSubcoreMesh(
        core_axis_name="c", subcore_axis_name="s",
        num_cores=NUM_CORES, num_subcores=NUM_SUBCORES)

    BBP = BB + 8
    NI = 4
    scratch = (
        [pltpu.VMEM((BB,), jnp.int32) for _ in range(4 * NI)]
        + [pltpu.VMEM((BB, d), jnp.float32) for _ in range(NBUF) for d in _DIMS]
        + [pltpu.VMEM((DT, 8, BBP), jnp.float32) for _ in range(NBUF)]
        + [pltpu.SemaphoreType.DMA for _ in range(NI + 2 * NBUF)]
    )

    @functools.partial(
        pl.kernel,
        mesh=mesh,
        compiler_params=pltpu.CompilerParams(use_tc_tiling_on_sc=False,
                                             needs_layout_passes=False),
        out_type=jax.ShapeDtypeStruct((L, DT, n_bt, 8, BB), jnp.float32),
        scratch_types=scratch,
    )
    def fused_embed(word_h, p0_h, p1_h, p2_h, wt_h, t0_h, t1_h, t2_h,
                    out_h, *scr):
        idx_bufs = [scr[4 * q:4 * q + 4] for q in range(NI)]
        o = 4 * NI
        row_bufs = [scr[o + 4 * k:o + 4 * k + 4] for k in range(NBUF)]
        o += 4 * NBUF
        tbufs = scr[o:o + NBUF]
        o += NBUF
        sem_i = scr[o:o + NI]
        sem_g = scr[o + NI:o + NI + NBUF]
        sem_o = scr[o + NI + NBUF:o + NI + 2 * NBUF]
        srcs = (word_h, p0_h, p1_h, p2_h)
        tabs = (wt_h, t0_h, t1_h, t2_h)

        wid = lax.axis_index("s") * NUM_CORES + lax.axis_index("c")
        u0 = wid * units_per_w
        iota = lax.iota(jnp.int32, 16)
        dvecs = [j * 16 + iota for j in range(OUT_DIM // 16)]
        i0s = [d // 8 for d in dvecs]
        i1s = [d % 8 for d in dvecs]

        def idx_issue(u, q):
            base = jnp.minimum(u, u0 + units_per_w - 1) * BB
            for s, d in zip(srcs, idx_bufs[q]):
                pltpu.async_copy(s.at[pl.ds(base, BB)], d, sem_i[q])

        def idx_wait(q):
            for s, d in zip(srcs, idx_bufs[q]):
                pltpu.make_async_copy(s.at[pl.ds(0, BB)], d, sem_i[q]).wait()

        def gather_issue(k, q):
            for tab, iv, b in zip(tabs, idx_bufs[q], row_bufs[k]):
                pltpu.async_copy(tab.at[iv], b, sem_g[k])

        def gather_wait(k, q):
            for tab, iv, b in zip(tabs, idx_bufs[q], row_bufs[k]):
                pltpu.make_async_copy(tab.at[iv], b, sem_g[k]).wait()

        def write_issue(u, k):
            l, bt = u // n_bt, u % n_bt
            pltpu.async_copy(tbufs[k].at[:, :, pl.ds(0, BB)], out_h.at[l, :, bt],
                             sem_o[k])

        def write_wait(k):
            pltpu.make_async_copy(tbufs[k].at[:, :, pl.ds(0, BB)],
                                  out_h.at[0, :, 0], sem_o[k]).wait()

        def transpose_unit(k, q):
            bw, b0, b1, b2 = row_bufs[k]
            tb = tbufs[k]
            nw = WORD_DIM // 16

            def t_body(th, carry):
                for dt in range(2):
                    t = th * 2 + dt
                    tvec = jnp.full((16,), t, jnp.int32)
                    for j in range(nw):
                        vals = bw[t, pl.ds(j * 16, 16)]
                        plsc.store_scatter(tb, [i0s[j], i1s[j], tvec], vals)
                    for pi, bp in enumerate((b0, b1, b2)):
                        j = nw + pi
                        vals = bp[t, pl.ds(0, POS_DIM)]
                        plsc.store_scatter(tb, [i0s[j], i1s[j], tvec], vals)
                return carry

            lax.fori_loop(0, BB // 2, t_body, 0)

        for q in range(NI):
            idx_issue(u0 + q, q)
        for u in range(NBUF):
            idx_wait(u)
            gather_issue(u, u)

        def body(t, carry):
            u_base = u0 + t * NI
            for j in range(NI):
                u = u_base + j
                k = j % NBUF
                gather_wait(k, j)
                if j >= NBUF:
                    write_wait(k)
                else:
                    @pl.when(t > 0)
                    def _(k=k):
                        write_wait(k)
                idx_issue(u + NI, j)
                transpose_unit(k, j)
                write_issue(u, k)
                idx_wait((j + NBUF) % NI)
                gather_issue(k, (j + NBUF) % NI)
            return carry

        lax.fori_loop(0, units_per_w // NI, body, 0)
        for q in range(NBUF):
            idx_wait((q + NBUF) % NI)
            gather_wait(q, q)
            write_wait(q)

    return fused_embed


def kernel(word, pos0, pos1, pos2, word_table, pos0_table, pos1_table, pos2_table):
    B, L = word.shape
    fused = _build(B, L)
    X = fused(word.T.reshape(B * L), pos0.T.reshape(B * L),
              pos1.T.reshape(B * L), pos2.T.reshape(B * L),
              word_table, pos0_table, pos1_table, pos2_table)
    return X.transpose(2, 4, 0, 1, 3).reshape(B, L, OUT_DIM)
